# baseline (speedup 1.0000x reference)
import functools

import jax
import jax.numpy as jnp
from jax import lax
from jax.experimental import pallas as pl
from jax.experimental.pallas import tpu as pltpu
from jax.experimental.pallas import tpu_sc as plsc

NC = 2
NS = 16
NW = NC * NS
L = 16
GCHUNK = 128


def _sc_loss(y_h, x_h, x1y_h, x1x_h, m1_h, m2_h, pd_h, out_h,
             yv, xv, f1y, f1x, mm1, mm2, idx, idy, vdx, vdy, part,
             sem_in, sem_g, *, P, N, Hd, Wd):
    wid = lax.axis_index("s") * NC + lax.axis_index("c")
    base = wid * P
    b = base // N
    plane = Hd * Wd
    bias_dx = b * (2 * plane)

    cps = [
        pltpu.async_copy(y_h.at[pl.ds(base, P)], yv, sem_in),
        pltpu.async_copy(x_h.at[pl.ds(base, P)], xv, sem_in),
        pltpu.async_copy(x1y_h.at[pl.ds(base, P)], f1y, sem_in),
        pltpu.async_copy(x1x_h.at[pl.ds(base, P)], f1x, sem_in),
        pltpu.async_copy(m1_h.at[pl.ds(base, P)], mm1, sem_in),
        pltpu.async_copy(m2_h.at[pl.ds(base, P)], mm2, sem_in),
    ]
    for c in cps:
        c.wait()

    kpr = GCHUNK // L

    def body1(i, carry):
        off = pl.multiple_of(i * L, L)
        yc = jnp.clip(yv[pl.ds(off, L)], 0, Hd - 1)
        xc = jnp.clip(xv[pl.ds(off, L)], 0, Wd - 1)
        lin = bias_dx + yc * Wd + xc
        j = i // kpr
        koff = pl.multiple_of((i % kpr) * L, L)
        idx[j, pl.ds(koff, L)] = lin
        idy[j, pl.ds(koff, L)] = lin + plane
        return carry

    lax.fori_loop(0, P // L, body1, 0)

    descs = []
    for j in range(P // GCHUNK):
        descs.append(pltpu.async_copy(pd_h.at[idx.at[j]], vdx.at[j], sem_g))
        descs.append(pltpu.async_copy(pd_h.at[idy.at[j]], vdy.at[j], sem_g))
    for d in descs:
        d.wait()

    def body2(i, acc):
        off = pl.multiple_of(i * L, L)
        j = i // kpr
        koff = pl.multiple_of((i % kpr) * L, L)
        dxv = vdx[j, pl.ds(koff, L)]
        dyv = vdy[j, pl.ds(koff, L)]
        yc = yv[pl.ds(off, L)].astype(jnp.float32)
        xc = xv[pl.ds(off, L)].astype(jnp.float32)
        m = (mm1[pl.ds(off, L)] * mm2[pl.ds(off, L)]).astype(jnp.float32)
        e0 = yc + dxv - f1y[pl.ds(off, L)]
        e1 = xc + dyv - f1x[pl.ds(off, L)]
        return acc + m * (e0 * e0 + e1 * e1)

    acc = lax.fori_loop(0, P // L, body2, jnp.zeros((L,), jnp.float32))
    part[...] = acc
    pltpu.sync_copy(part, out_h.at[wid])


def kernel(x1, x2, kp1_mask, kp2_mask, pixel_delta, H, W):
    B, N, _ = x2.shape
    _, _, Hd, Wd = pixel_delta.shape
    P = (B * N) // NW
    assert (B * N) % NW == 0 and N % P == 0 and P % GCHUNK == 0

    y_idx = x2[:, :, 0].reshape(-1)
    x_idx = x2[:, :, 1].reshape(-1)
    x1y = x1[:, :, 0].reshape(-1)
    x1x = x1[:, :, 1].reshape(-1)
    m1 = kp1_mask.reshape(-1).astype(jnp.int32)
    m2 = kp2_mask.reshape(-1).astype(jnp.int32)
    one = lax.optimization_barrier(jnp.float32(1.0))
    pd = pixel_delta.reshape(-1) * one

    mesh = plsc.VectorSubcoreMesh(
        core_axis_name="c", subcore_axis_name="s",
        num_cores=NC, num_subcores=NS)
    G = P // GCHUNK

    run = functools.partial(
        pl.kernel,
        out_type=jax.ShapeDtypeStruct((NW, L), jnp.float32),
        mesh=mesh,
        scratch_types=[
            pltpu.VMEM((P,), jnp.int32),
            pltpu.VMEM((P,), jnp.int32),
            pltpu.VMEM((P,), jnp.float32),
            pltpu.VMEM((P,), jnp.float32),
            pltpu.VMEM((P,), jnp.int32),
            pltpu.VMEM((P,), jnp.int32),
            pltpu.VMEM((G, GCHUNK), jnp.int32),
            pltpu.VMEM((G, GCHUNK), jnp.int32),
            pltpu.VMEM((G, GCHUNK), jnp.float32),
            pltpu.VMEM((G, GCHUNK), jnp.float32),
            pltpu.VMEM((L,), jnp.float32),
            pltpu.SemaphoreType.DMA,
            pltpu.SemaphoreType.DMA,
        ],
    )(functools.partial(_sc_loss, P=P, N=N, Hd=Hd, Wd=Wd))

    partials = run(y_idx, x_idx, x1y, x1x, m1, m2, pd)
    return jnp.sum(partials) / (B * N * 2)


# speedup vs baseline: 1.1461x; 1.1461x over previous
import functools

import jax
import jax.numpy as jnp
from jax import lax
from jax.experimental import pallas as pl
from jax.experimental.pallas import tpu as pltpu
from jax.experimental.pallas import tpu_sc as plsc

NC = 2
NS = 16
NW = NC * NS
L = 16
GCHUNK = 128


def _sc_loss(y_h, x_h, x1y_h, x1x_h, m1_h, m2_h, pd_h, out_h,
             yv, xv, f1y, f1x, mm1, mm2, idx, idy, vdx, vdy, part,
             sem_in, sem_g, *, P, N, Hd, Wd):
    wid = lax.axis_index("s") * NC + lax.axis_index("c")
    base = wid * P
    b = base // N
    plane = Hd * Wd
    bias_dx = b * (2 * plane)

    cps = [
        pltpu.async_copy(y_h.at[pl.ds(base, P)], yv, sem_in),
        pltpu.async_copy(x_h.at[pl.ds(base, P)], xv, sem_in),
        pltpu.async_copy(x1y_h.at[pl.ds(base, P)], f1y, sem_in),
        pltpu.async_copy(x1x_h.at[pl.ds(base, P)], f1x, sem_in),
        pltpu.async_copy(m1_h.at[pl.ds(base, P)], mm1, sem_in),
        pltpu.async_copy(m2_h.at[pl.ds(base, P)], mm2, sem_in),
    ]
    for c in cps:
        c.wait()

    kpr = GCHUNK // L

    def body1(i, carry):
        off = pl.multiple_of(i * L, L)
        yc = jnp.clip(yv[pl.ds(off, L)], 0, Hd - 1)
        xc = jnp.clip(xv[pl.ds(off, L)], 0, Wd - 1)
        lin = bias_dx + yc * Wd + xc
        j = i // kpr
        koff = pl.multiple_of((i % kpr) * L, L)
        idx[j, pl.ds(koff, L)] = lin
        idy[j, pl.ds(koff, L)] = lin + plane
        return carry

    descs = []
    for j in range(P // GCHUNK):
        lax.fori_loop(j * kpr, (j + 1) * kpr, body1, 0)
        descs.append(pltpu.async_copy(pd_h.at[idx.at[j]], vdx.at[j], sem_g))
        descs.append(pltpu.async_copy(pd_h.at[idy.at[j]], vdy.at[j], sem_g))

    def body2(i, acc):
        off = pl.multiple_of(i * L, L)
        j = i // kpr
        koff = pl.multiple_of((i % kpr) * L, L)
        dxv = vdx[j, pl.ds(koff, L)]
        dyv = vdy[j, pl.ds(koff, L)]
        yc = yv[pl.ds(off, L)].astype(jnp.float32)
        xc = xv[pl.ds(off, L)].astype(jnp.float32)
        m = (mm1[pl.ds(off, L)] * mm2[pl.ds(off, L)]).astype(jnp.float32)
        e0 = yc + dxv - f1y[pl.ds(off, L)]
        e1 = xc + dyv - f1x[pl.ds(off, L)]
        return acc + m * (e0 * e0 + e1 * e1)

    acc = jnp.zeros((L,), jnp.float32)
    for j in range(P // GCHUNK):
        descs[2 * j].wait()
        descs[2 * j + 1].wait()
        acc = lax.fori_loop(j * kpr, (j + 1) * kpr, body2, acc)
    part[...] = acc
    pltpu.sync_copy(part, out_h.at[wid])


def kernel(x1, x2, kp1_mask, kp2_mask, pixel_delta, H, W):
    B, N, _ = x2.shape
    _, _, Hd, Wd = pixel_delta.shape
    P = (B * N) // NW
    assert (B * N) % NW == 0 and N % P == 0 and P % GCHUNK == 0

    y_idx = x2[:, :, 0].reshape(-1)
    x_idx = x2[:, :, 1].reshape(-1)
    x1y = x1[:, :, 0].reshape(-1)
    x1x = x1[:, :, 1].reshape(-1)
    m1 = kp1_mask.reshape(-1).astype(jnp.int32)
    m2 = kp2_mask.reshape(-1).astype(jnp.int32)
    pd = pixel_delta.reshape(-1)

    mesh = plsc.VectorSubcoreMesh(
        core_axis_name="c", subcore_axis_name="s",
        num_cores=NC, num_subcores=NS)
    G = P // GCHUNK

    run = functools.partial(
        pl.kernel,
        out_type=jax.ShapeDtypeStruct((NW, L), jnp.float32),
        mesh=mesh,
        scratch_types=[
            pltpu.VMEM((P,), jnp.int32),
            pltpu.VMEM((P,), jnp.int32),
            pltpu.VMEM((P,), jnp.float32),
            pltpu.VMEM((P,), jnp.float32),
            pltpu.VMEM((P,), jnp.int32),
            pltpu.VMEM((P,), jnp.int32),
            pltpu.VMEM((G, GCHUNK), jnp.int32),
            pltpu.VMEM((G, GCHUNK), jnp.int32),
            pltpu.VMEM((G, GCHUNK), jnp.float32),
            pltpu.VMEM((G, GCHUNK), jnp.float32),
            pltpu.VMEM((L,), jnp.float32),
            pltpu.SemaphoreType.DMA,
            pltpu.SemaphoreType.DMA,
        ],
    )(functools.partial(_sc_loss, P=P, N=N, Hd=Hd, Wd=Wd))

    partials = run(y_idx, x_idx, x1y, x1x, m1, m2, pd)
    return jnp.sum(partials) / (B * N * 2)


# speedup vs baseline: 1.3512x; 1.1789x over previous
import functools

import jax
import jax.numpy as jnp
from jax import lax
from jax.experimental import pallas as pl
from jax.experimental.pallas import tpu as pltpu
from jax.experimental.pallas import tpu_sc as plsc

NC = 2
NS = 16
NW = NC * NS
L = 16
GCHUNK = 128


def _sc_loss(y_h, x_h, x1y_h, x1x_h, m1_h, m2_h, pd_h, out_h,
             yv, xv, f1y, f1x, mm1, mm2, idx, idy, vdx, vdy, part,
             sem_in, sem_g, *, P, N, Hd, Wd):
    wid = lax.axis_index("s") * NC + lax.axis_index("c")
    base = wid * P
    b = base // N
    plane = Hd * Wd
    bias_dx = b * (2 * plane)

    cps = [
        pltpu.async_copy(y_h.at[pl.ds(base, P)], yv, sem_in),
        pltpu.async_copy(x_h.at[pl.ds(base, P)], xv, sem_in),
        pltpu.async_copy(x1y_h.at[pl.ds(base, P)], f1y, sem_in),
        pltpu.async_copy(x1x_h.at[pl.ds(base, P)], f1x, sem_in),
        pltpu.async_copy(m1_h.at[pl.ds(base, P)], mm1, sem_in),
        pltpu.async_copy(m2_h.at[pl.ds(base, P)], mm2, sem_in),
    ]
    for c in cps:
        c.wait()

    kpr = GCHUNK // L

    def body1(i, carry):
        off = pl.multiple_of(i * L, L)
        yc = jnp.clip(yv[pl.ds(off, L)], 0, Hd - 1)
        xc = jnp.clip(xv[pl.ds(off, L)], 0, Wd - 1)
        lin = bias_dx + yc * Wd + xc
        j = i // kpr
        koff = pl.multiple_of((i % kpr) * L, L)
        idx[j, pl.ds(koff, L)] = lin
        idy[j, pl.ds(koff, L)] = lin + plane
        return carry

    lax.fori_loop(0, P // L, body1, 0)

    descs = []
    for j in range(P // GCHUNK):
        descs.append(pltpu.async_copy(pd_h.at[idx.at[j]], vdx.at[j], sem_g))
        descs.append(pltpu.async_copy(pd_h.at[idy.at[j]], vdy.at[j], sem_g))
    for d in descs:
        d.wait()

    def body2(i, acc):
        off = pl.multiple_of(i * L, L)
        j = i // kpr
        koff = pl.multiple_of((i % kpr) * L, L)
        dxv = vdx[j, pl.ds(koff, L)]
        dyv = vdy[j, pl.ds(koff, L)]
        yc = yv[pl.ds(off, L)].astype(jnp.float32)
        xc = xv[pl.ds(off, L)].astype(jnp.float32)
        m = (mm1[pl.ds(off, L)] * mm2[pl.ds(off, L)]).astype(jnp.float32)
        e0 = yc + dxv - f1y[pl.ds(off, L)]
        e1 = xc + dyv - f1x[pl.ds(off, L)]
        return acc + m * (e0 * e0 + e1 * e1)

    acc = lax.fori_loop(0, P // L, body2, jnp.zeros((L,), jnp.float32))
    part[...] = acc
    pltpu.sync_copy(part, out_h.at[wid])


def kernel(x1, x2, kp1_mask, kp2_mask, pixel_delta, H, W):
    B, N, _ = x2.shape
    _, _, Hd, Wd = pixel_delta.shape
    P = (B * N) // NW
    assert (B * N) % NW == 0 and N % P == 0 and P % GCHUNK == 0

    y_idx = x2[:, :, 0].reshape(-1)
    x_idx = x2[:, :, 1].reshape(-1)
    x1y = x1[:, :, 0].reshape(-1)
    x1x = x1[:, :, 1].reshape(-1)
    m1 = kp1_mask.reshape(-1).astype(jnp.int32)
    m2 = kp2_mask.reshape(-1).astype(jnp.int32)
    pd = pixel_delta.reshape(-1)

    mesh = plsc.VectorSubcoreMesh(
        core_axis_name="c", subcore_axis_name="s",
        num_cores=NC, num_subcores=NS)
    G = P // GCHUNK

    run = functools.partial(
        pl.kernel,
        out_type=jax.ShapeDtypeStruct((NW, L), jnp.float32),
        mesh=mesh,
        scratch_types=[
            pltpu.VMEM((P,), jnp.int32),
            pltpu.VMEM((P,), jnp.int32),
            pltpu.VMEM((P,), jnp.float32),
            pltpu.VMEM((P,), jnp.float32),
            pltpu.VMEM((P,), jnp.int32),
            pltpu.VMEM((P,), jnp.int32),
            pltpu.VMEM((G, GCHUNK), jnp.int32),
            pltpu.VMEM((G, GCHUNK), jnp.int32),
            pltpu.VMEM((G, GCHUNK), jnp.float32),
            pltpu.VMEM((G, GCHUNK), jnp.float32),
            pltpu.VMEM((L,), jnp.float32),
            pltpu.SemaphoreType.DMA,
            pltpu.SemaphoreType.DMA,
        ],
    )(functools.partial(_sc_loss, P=P, N=N, Hd=Hd, Wd=Wd))

    partials = run(y_idx, x_idx, x1y, x1x, m1, m2, pd)
    return jnp.sum(partials) / (B * N * 2)


# speedup vs baseline: 1.3568x; 1.0042x over previous
import functools

import jax
import jax.numpy as jnp
from jax import lax
from jax.experimental import pallas as pl
from jax.experimental.pallas import tpu as pltpu
from jax.experimental.pallas import tpu_sc as plsc

NC = 2
NS = 16
NW = NC * NS
L = 16
GCHUNK = 128


def _sc_loss(y_h, x_h, x1y_h, x1x_h, m1_h, m2_h, pd_h, out_h,
             yv, xv, f1y, f1x, mm1, mm2, idx, idy, vdx, vdy, part,
             sem_in, sem_g, *, P, N, Hd, Wd):
    wid = lax.axis_index("s") * NC + lax.axis_index("c")
    base = wid * P
    b = base // N
    plane = Hd * Wd
    bias_dx = b * (2 * plane)

    cps = [
        pltpu.async_copy(y_h.at[pl.ds(base, P)], yv, sem_in),
        pltpu.async_copy(x_h.at[pl.ds(base, P)], xv, sem_in),
        pltpu.async_copy(x1y_h.at[pl.ds(base, P)], f1y, sem_in),
        pltpu.async_copy(x1x_h.at[pl.ds(base, P)], f1x, sem_in),
        pltpu.async_copy(m1_h.at[pl.ds(base, P)], mm1, sem_in),
        pltpu.async_copy(m2_h.at[pl.ds(base, P)], mm2, sem_in),
    ]
    cps[0].wait()
    cps[1].wait()

    kpr = GCHUNK // L

    def body1(i, carry):
        off = pl.multiple_of(i * L, L)
        yc = jnp.clip(yv[pl.ds(off, L)], 0, Hd - 1)
        xc = jnp.clip(xv[pl.ds(off, L)], 0, Wd - 1)
        lin = bias_dx + yc * Wd + xc
        j = i // kpr
        koff = pl.multiple_of((i % kpr) * L, L)
        idx[j, pl.ds(koff, L)] = lin
        idy[j, pl.ds(koff, L)] = lin + plane
        return carry

    lax.fori_loop(0, P // L, body1, 0)

    descs = []
    for j in range(P // GCHUNK):
        descs.append(pltpu.async_copy(pd_h.at[idx.at[j]], vdx.at[j], sem_g))
        descs.append(pltpu.async_copy(pd_h.at[idy.at[j]], vdy.at[j], sem_g))
    for c in cps[2:]:
        c.wait()
    for d in descs:
        d.wait()

    def body2(i, acc):
        off = pl.multiple_of(i * L, L)
        j = i // kpr
        koff = pl.multiple_of((i % kpr) * L, L)
        dxv = vdx[j, pl.ds(koff, L)]
        dyv = vdy[j, pl.ds(koff, L)]
        yc = yv[pl.ds(off, L)].astype(jnp.float32)
        xc = xv[pl.ds(off, L)].astype(jnp.float32)
        m = (mm1[pl.ds(off, L)] * mm2[pl.ds(off, L)]).astype(jnp.float32)
        e0 = yc + dxv - f1y[pl.ds(off, L)]
        e1 = xc + dyv - f1x[pl.ds(off, L)]
        return acc + m * (e0 * e0 + e1 * e1)

    acc = lax.fori_loop(0, P // L, body2, jnp.zeros((L,), jnp.float32))
    part[...] = acc
    pltpu.sync_copy(part, out_h.at[wid])


def kernel(x1, x2, kp1_mask, kp2_mask, pixel_delta, H, W):
    B, N, _ = x2.shape
    _, _, Hd, Wd = pixel_delta.shape
    P = (B * N) // NW
    assert (B * N) % NW == 0 and N % P == 0 and P % GCHUNK == 0

    y_idx = x2[:, :, 0].reshape(-1)
    x_idx = x2[:, :, 1].reshape(-1)
    x1y = x1[:, :, 0].reshape(-1)
    x1x = x1[:, :, 1].reshape(-1)
    m1 = kp1_mask.reshape(-1).astype(jnp.int32)
    m2 = kp2_mask.reshape(-1).astype(jnp.int32)
    pd = pixel_delta.reshape(-1)

    mesh = plsc.VectorSubcoreMesh(
        core_axis_name="c", subcore_axis_name="s",
        num_cores=NC, num_subcores=NS)
    G = P // GCHUNK

    run = functools.partial(
        pl.kernel,
        out_type=jax.ShapeDtypeStruct((NW, L), jnp.float32),
        mesh=mesh,
        scratch_types=[
            pltpu.VMEM((P,), jnp.int32),
            pltpu.VMEM((P,), jnp.int32),
            pltpu.VMEM((P,), jnp.float32),
            pltpu.VMEM((P,), jnp.float32),
            pltpu.VMEM((P,), jnp.int32),
            pltpu.VMEM((P,), jnp.int32),
            pltpu.VMEM((G, GCHUNK), jnp.int32),
            pltpu.VMEM((G, GCHUNK), jnp.int32),
            pltpu.VMEM((G, GCHUNK), jnp.float32),
            pltpu.VMEM((G, GCHUNK), jnp.float32),
            pltpu.VMEM((L,), jnp.float32),
            pltpu.SemaphoreType.DMA,
            pltpu.SemaphoreType.DMA,
        ],
    )(functools.partial(_sc_loss, P=P, N=N, Hd=Hd, Wd=Wd))

    partials = run(y_idx, x_idx, x1y, x1x, m1, m2, pd)
    return jnp.sum(partials) / (B * N * 2)
